# baseline (speedup 1.0000x reference)
import functools

import jax
import jax.numpy as jnp
from jax import lax
from jax.experimental import pallas as pl
from jax.experimental.pallas import tpu as pltpu
from jax.experimental.pallas import tpu_sc as plsc

N = 1048576
GRID = 128
NFLAT = GRID * GRID * GRID
NW = 32
NPER = N // NW
C = 4096
NCHUNK = NPER // C
J = C // 128
L = 16

OFFS = (0, 1, 128, 129, 16384, 16385, 16512, 16513)


def _body(x_hbm, tbl_hbm, o_hbm, xbuf, fbuf, idxbuf, vbuf, obuf, sem):
    wid = lax.axis_index("s") * 2 + lax.axis_index("c")
    base_pt = wid * NPER

    iota = lax.iota(jnp.int32, L)
    iota3 = iota * 3
    scale = jnp.full((L,), 125.0, jnp.float32)
    one = jnp.full((L,), 1.0, jnp.float32)
    cols = [jnp.full((L,), c8, jnp.int32) for c8 in range(8)]

    def chunk_body(t, carry):
        off = base_pt + t * C
        pltpu.sync_copy(x_hbm.at[pl.ds(off * 3, C * 3)], xbuf)

        def row_fire(j, carry):
            for g in range(8):
                p = j * 128 + g * L
                pidx = iota3 + (3 * p)
                x0 = plsc.load_gather(xbuf, [pidx])
                x1 = plsc.load_gather(xbuf, [pidx + 1])
                x2 = plsc.load_gather(xbuf, [pidx + 2])
                t0 = x0 * scale
                t1 = x1 * scale
                t2 = x2 * scale
                i0 = jnp.clip(t0.astype(jnp.int32), 0, GRID - 2)
                i1 = jnp.clip(t1.astype(jnp.int32), 0, GRID - 2)
                i2 = jnp.clip(t2.astype(jnp.int32), 0, GRID - 2)
                fbuf[0, pl.ds(p, L)] = t0 - i0.astype(jnp.float32)
                fbuf[1, pl.ds(p, L)] = t1 - i1.astype(jnp.float32)
                fbuf[2, pl.ds(p, L)] = t2 - i2.astype(jnp.float32)
                flat = (
                    lax.shift_left(i0, 14) + lax.shift_left(i1, 7) + i2
                )
                idxbuf[j, pl.ds(g * L, L)] = flat
            pltpu.async_copy(
                tbl_hbm.at[idxbuf.at[j]],
                vbuf.at[pl.ds(j * 128, 128), :],
                sem,
            )
            return carry

        lax.fori_loop(0, J, row_fire, 0, unroll=False)

        def row_drain(j, carry):
            pltpu.make_async_copy(
                tbl_hbm.at[idxbuf.at[j]],
                vbuf.at[pl.ds(j * 128, 128), :],
                sem,
            ).wait()
            return carry

        lax.fori_loop(0, J, row_drain, 0, unroll=False)

        def row_acc(j, carry):
            for g in range(8):
                p = j * 128 + g * L
                pidx = iota + p
                f0 = fbuf[0, pl.ds(p, L)]
                f1 = fbuf[1, pl.ds(p, L)]
                f2 = fbuf[2, pl.ds(p, L)]
                g0 = one - f0
                g1 = one - f1
                g2 = one - f2
                acc = None
                for c8 in range(8):
                    wx = f0 if (c8 & 4) else g0
                    wy = f1 if (c8 & 2) else g1
                    wz = f2 if (c8 & 1) else g2
                    w = wx * wy * wz
                    v = plsc.load_gather(vbuf, [pidx, cols[c8]])
                    term = w * v
                    acc = term if acc is None else acc + term
                obuf[pl.ds(p, L)] = acc
            return carry

        lax.fori_loop(0, J, row_acc, 0, unroll=False)

        pltpu.sync_copy(obuf, o_hbm.at[pl.ds(off, C)])
        return carry

    lax.fori_loop(0, NCHUNK, chunk_body, 0, unroll=False)


@jax.jit
def _run(x, tbl):
    mesh = plsc.VectorSubcoreMesh(core_axis_name="c", subcore_axis_name="s")
    kern = pl.kernel(
        _body,
        out_type=jax.ShapeDtypeStruct((N,), jnp.float32),
        mesh=mesh,
        scratch_types=[
            pltpu.VMEM((C * 3,), jnp.float32),
            pltpu.VMEM((3, C), jnp.float32),
            pltpu.VMEM((J, 128), jnp.int32),
            pltpu.VMEM((C, 8), jnp.float32),
            pltpu.VMEM((C,), jnp.float32),
            pltpu.SemaphoreType.DMA,
        ],
        compiler_params=pltpu.CompilerParams(
            needs_layout_passes=False, use_tc_tiling_on_sc=False
        ),
    )
    return kern(x, tbl)


@jax.jit
def _build_table(grid):
    gf = grid.reshape(-1)
    gfp = jnp.concatenate([gf, jnp.zeros((16520,), jnp.float32)])
    return jnp.stack([gfp[o : o + NFLAT] for o in OFFS], axis=1)


def kernel(x, grid):
    return _run(x.reshape(-1), _build_table(grid)).reshape(N, 1)


# speedup vs baseline: 1.5000x; 1.5000x over previous
import functools

import jax
import jax.numpy as jnp
from jax import lax
from jax.experimental import pallas as pl
from jax.experimental.pallas import tpu as pltpu
from jax.experimental.pallas import tpu_sc as plsc

N = 1048576
GRID = 128
NFLAT = GRID * GRID * GRID
NW = 32
L = 16

OFFS = (0, 1, 128, 129, 16384, 16385, 16512, 16513)

TB = 2048
TBLK = NFLAT // NW // TB

NPER = N // NW
C = 4096
NCHUNK = NPER // C
J = C // 128


def _wid():
    return lax.axis_index("s") * 2 + lax.axis_index("c")


def _build_body(gf_hbm, tbl_hbm, winbuf, outbuf, si0, si1, so0, so1):
    base = _wid() * (NFLAT // NW)
    iota = lax.iota(jnp.int32, L)
    cols = [jnp.full((L,), c8, jnp.int32) for c8 in range(8)]
    sin = (si0, si1)
    sout = (so0, so1)
    WOFFS = (0, 128, 16384, 16512)
    TBW = TB + L

    def fire_in(b, s):
        k0 = base + b * TB
        for m, o in enumerate(WOFFS):
            start = jnp.minimum(k0 + o, NFLAT - TBW)
            pltpu.async_copy(
                gf_hbm.at[pl.ds(start, TBW)], winbuf.at[s, m], sin[s]
            )

    def drain_in(b, s):
        k0 = base + b * TB
        for m, o in enumerate(WOFFS):
            start = jnp.minimum(k0 + o, NFLAT - TBW)
            pltpu.make_async_copy(
                gf_hbm.at[pl.ds(start, TBW)], winbuf.at[s, m], sin[s]
            ).wait()

    def out_copy(b, s):
        k0 = base + b * TB
        return pltpu.make_async_copy(
            outbuf.at[s], tbl_hbm.at[pl.ds(k0, TB), :], sout[s]
        )

    fire_in(0, 0)

    def pair(i, carry):
        for s in (0, 1):
            b = 2 * i + s

            @pl.when(b + 1 < TBLK)
            def _():
                fire_in(b + 1, 1 - s)

            drain_in(b, s)

            @pl.when(b >= 2)
            def _():
                out_copy(b - 2, s).wait()

            def grp(g, carry):
                for u in range(8):
                    p = (g * 8 + u) * L
                    pidx = iota + p
                    for m in range(4):
                        ve = winbuf[s, m, pl.ds(p, L)]
                        vo = plsc.load_gather(
                            winbuf.at[s, m], [pidx + 1]
                        )
                        plsc.store_scatter(
                            outbuf.at[s], [pidx, cols[2 * m]], ve
                        )
                        plsc.store_scatter(
                            outbuf.at[s], [pidx, cols[2 * m + 1]], vo
                        )
                return carry

            lax.fori_loop(0, TB // L // 8, grp, 0, unroll=False)
            out_copy(b, s).start()
        return carry

    lax.fori_loop(0, TBLK // 2, pair, 0, unroll=False)
    out_copy(TBLK - 2, 0).wait()
    out_copy(TBLK - 1, 1).wait()


def _gather_body(x_hbm, tbl_hbm, o_hbm, xbuf, fbuf, idxbuf, vbuf, obuf, sem):
    base_pt = _wid() * NPER

    iota = lax.iota(jnp.int32, L)
    iota3 = iota * 3
    scale = jnp.full((L,), 125.0, jnp.float32)
    one = jnp.full((L,), 1.0, jnp.float32)
    cols = [jnp.full((L,), c8, jnp.int32) for c8 in range(8)]

    def chunk_body(t, carry):
        off = base_pt + t * C
        pltpu.sync_copy(x_hbm.at[pl.ds(off * 3, C * 3)], xbuf)

        def row_fire(j, carry):
            for g in range(8):
                p = j * 128 + g * L
                pidx = iota3 + (3 * p)
                x0 = plsc.load_gather(xbuf, [pidx])
                x1 = plsc.load_gather(xbuf, [pidx + 1])
                x2 = plsc.load_gather(xbuf, [pidx + 2])
                t0 = x0 * scale
                t1 = x1 * scale
                t2 = x2 * scale
                i0 = jnp.clip(t0.astype(jnp.int32), 0, GRID - 3)
                i1 = jnp.clip(t1.astype(jnp.int32), 0, GRID - 3)
                i2 = jnp.clip(t2.astype(jnp.int32), 0, GRID - 3)
                fbuf[0, pl.ds(p, L)] = t0 - i0.astype(jnp.float32)
                fbuf[1, pl.ds(p, L)] = t1 - i1.astype(jnp.float32)
                fbuf[2, pl.ds(p, L)] = t2 - i2.astype(jnp.float32)
                flat = (
                    lax.shift_left(i0, 14) + lax.shift_left(i1, 7) + i2
                )
                idxbuf[j, pl.ds(g * L, L)] = flat
            pltpu.async_copy(
                tbl_hbm.at[idxbuf.at[j]],
                vbuf.at[pl.ds(j * 128, 128), :],
                sem,
            )
            return carry

        lax.fori_loop(0, J, row_fire, 0, unroll=False)

        def row_drain(j, carry):
            pltpu.make_async_copy(
                tbl_hbm.at[idxbuf.at[j]],
                vbuf.at[pl.ds(j * 128, 128), :],
                sem,
            ).wait()
            return carry

        lax.fori_loop(0, J, row_drain, 0, unroll=False)

        def row_acc(j, carry):
            for g in range(8):
                p = j * 128 + g * L
                pidx = iota + p
                f0 = fbuf[0, pl.ds(p, L)]
                f1 = fbuf[1, pl.ds(p, L)]
                f2 = fbuf[2, pl.ds(p, L)]
                g0 = one - f0
                g1 = one - f1
                g2 = one - f2
                acc = None
                for c8 in range(8):
                    wx = f0 if (c8 & 4) else g0
                    wy = f1 if (c8 & 2) else g1
                    wz = f2 if (c8 & 1) else g2
                    w = wx * wy * wz
                    v = plsc.load_gather(vbuf, [pidx, cols[c8]])
                    term = w * v
                    acc = term if acc is None else acc + term
                obuf[pl.ds(p, L)] = acc
            return carry

        lax.fori_loop(0, J, row_acc, 0, unroll=False)

        pltpu.sync_copy(obuf, o_hbm.at[pl.ds(off, C)])
        return carry

    lax.fori_loop(0, NCHUNK, chunk_body, 0, unroll=False)


_SC_PARAMS = pltpu.CompilerParams(
    needs_layout_passes=False, use_tc_tiling_on_sc=False
)


@jax.jit
def _run(x, gf):
    mesh = plsc.VectorSubcoreMesh(core_axis_name="c", subcore_axis_name="s")
    build = pl.kernel(
        _build_body,
        out_type=jax.ShapeDtypeStruct((NFLAT, 8), jnp.float32),
        mesh=mesh,
        scratch_types=[
            pltpu.VMEM((2, 4, TB + L), jnp.float32),
            pltpu.VMEM((2, TB, 8), jnp.float32),
            pltpu.SemaphoreType.DMA,
            pltpu.SemaphoreType.DMA,
            pltpu.SemaphoreType.DMA,
            pltpu.SemaphoreType.DMA,
        ],
        compiler_params=_SC_PARAMS,
    )
    tbl = build(gf)
    gather = pl.kernel(
        _gather_body,
        out_type=jax.ShapeDtypeStruct((N,), jnp.float32),
        mesh=mesh,
        scratch_types=[
            pltpu.VMEM((C * 3,), jnp.float32),
            pltpu.VMEM((3, C), jnp.float32),
            pltpu.VMEM((J, 128), jnp.int32),
            pltpu.VMEM((C, 8), jnp.float32),
            pltpu.VMEM((C,), jnp.float32),
            pltpu.SemaphoreType.DMA,
        ],
        compiler_params=_SC_PARAMS,
    )
    return gather(x, tbl)


def kernel(x, grid):
    return _run(x.reshape(-1), grid.reshape(-1)).reshape(N, 1)
